# baseline (speedup 1.0000x reference)
import jax
import jax.numpy as jnp
from jax import lax
from jax.experimental import pallas as pl
from jax.experimental.pallas import tpu as pltpu
from jax.experimental.pallas import tpu_sc as plsc

VOCAB = 1000000
EMBED_DIM = 32
B = 16384
L = 50

NC = 2
NS = 16
NW = NC * NS

RB = 32
CH = RB * L
ROWS_PER_W = B // NW
NCHUNK = ROWS_PER_W // RB
LPAD = 56


def _emb_body(idx_hbm, table_hbm, out_hbm, idx_v, rows_v,
              gsem0, gsem1, ssem0, ssem1):
  c = lax.axis_index("c")
  s = lax.axis_index("s")
  wid = s * NC + c
  base = wid * ROWS_PER_W
  gsems = (gsem0, gsem1)
  ssems = (ssem0, ssem1)

  def fire(i, b):
    pltpu.sync_copy(idx_hbm.at[pl.ds(base + i * RB, RB), :], idx_v.at[b])
    for r in range(RB):
      pltpu.async_copy(
          table_hbm.at[idx_v.at[b, r]],
          rows_v.at[b, r],
          gsems[b],
      )

  def drain(b):
    for r in range(RB):
      pltpu.make_async_copy(
          table_hbm.at[idx_v.at[b, r]],
          rows_v.at[b, r],
          gsems[b],
      ).wait()

  def store(i, b):
    pltpu.async_copy(
        rows_v.at[b], out_hbm.at[pl.ds(base + i * RB, RB)], ssems[b])

  def wait_store(i, b):
    pltpu.make_async_copy(
        rows_v.at[b], out_hbm.at[pl.ds(base + i * RB, RB)], ssems[b]).wait()

  fire(0, 0)
  fire(1, 1)

  def outer(i):
    drain(0)
    store(i, 0)

    @pl.when(i + 2 < NCHUNK)
    def _():
      wait_store(i, 0)
      fire(i + 2, 0)

    drain(1)
    store(i + 1, 1)

    @pl.when(i + 3 < NCHUNK)
    def _():
      wait_store(i + 1, 1)
      fire(i + 3, 1)

  pl.loop(0, NCHUNK, step=2)(outer)
  wait_store(NCHUNK - 2, 0)
  wait_store(NCHUNK - 1, 1)


@jax.jit
def _embedding_sc(batch, weight):
  mesh = plsc.VectorSubcoreMesh(core_axis_name="c", subcore_axis_name="s")
  out = pl.kernel(
      _emb_body,
      out_type=jax.ShapeDtypeStruct((B, L, EMBED_DIM), jnp.float32),
      mesh=mesh,
      scratch_types=[
          pltpu.VMEM((2, RB, L), jnp.int32),
          pltpu.VMEM((2, RB, L, EMBED_DIM), jnp.float32),
          pltpu.SemaphoreType.DMA,
          pltpu.SemaphoreType.DMA,
          pltpu.SemaphoreType.DMA,
          pltpu.SemaphoreType.DMA,
      ],
      compiler_params=pltpu.CompilerParams(use_tc_tiling_on_sc=False),
  )(batch, weight)
  return out


def kernel(batch, weight):
  return _embedding_sc(batch, weight)


# speedup vs baseline: 1.0545x; 1.0545x over previous
import jax
import jax.numpy as jnp
from jax import lax
from jax.experimental import pallas as pl
from jax.experimental.pallas import tpu as pltpu
from jax.experimental.pallas import tpu_sc as plsc

VOCAB = 1000000
EMBED_DIM = 32
B = 16384
L = 50

NC = 2
NS = 16
NW = NC * NS

BSLAB = B // NW
G = BSLAB // 128

_MESH = plsc.VectorSubcoreMesh(core_axis_name="c", subcore_axis_name="s")


def _stage_idx_body(idxt_hbm, idxl_hbm, buf8, buf2):
  c = lax.axis_index("c")
  s = lax.axis_index("s")
  wid = s * NC + c
  b0 = wid * BSLAB
  for l0 in range(0, 48, 8):
    pltpu.sync_copy(idxt_hbm.at[pl.ds(l0, 8), pl.ds(b0, BSLAB)], buf8)
    for r in range(8):
      pltpu.sync_copy(buf8.at[r],
                      idxl_hbm.at[pl.ds((l0 + r) * B + b0, BSLAB)])
  pltpu.sync_copy(idxt_hbm.at[pl.ds(48, 2), pl.ds(b0, BSLAB)], buf2)
  for r in range(2):
    pltpu.sync_copy(buf2.at[r],
                    idxl_hbm.at[pl.ds((48 + r) * B + b0, BSLAB)])


def _gather_body(idxl_hbm, table_hbm, out_hbm, idx_v, rows_v,
                 gsem0, gsem1, ssem0, ssem1):
  c = lax.axis_index("c")
  s = lax.axis_index("s")
  wid = s * NC + c
  b0 = wid * BSLAB
  gsems = (gsem0, gsem1)
  ssems = (ssem0, ssem1)

  def fire(l, b):
    pltpu.sync_copy(idxl_hbm.at[pl.ds(l * B + b0, BSLAB)], idx_v.at[b])
    for j in range(G):
      pltpu.async_copy(
          table_hbm.at[idx_v.at[b, pl.ds(j * 128, 128)]],
          rows_v.at[b, pl.ds(j * 128, 128)],
          gsems[b],
      )

  def drain(b):
    for j in range(G):
      pltpu.make_async_copy(
          table_hbm.at[idx_v.at[b, pl.ds(j * 128, 128)]],
          rows_v.at[b, pl.ds(j * 128, 128)],
          gsems[b],
      ).wait()

  def store(l, b):
    pltpu.async_copy(
        rows_v.at[b], out_hbm.at[l, pl.ds(b0, BSLAB)], ssems[b])

  def wait_store(l, b):
    pltpu.make_async_copy(
        rows_v.at[b], out_hbm.at[l, pl.ds(b0, BSLAB)], ssems[b]).wait()

  fire(0, 0)
  fire(1, 1)

  def outer(l):
    drain(0)
    store(l, 0)

    @pl.when(l + 2 < L)
    def _():
      wait_store(l, 0)
      fire(l + 2, 0)

    drain(1)
    store(l + 1, 1)

    @pl.when(l + 3 < L)
    def _():
      wait_store(l + 1, 1)
      fire(l + 3, 1)

  pl.loop(0, L, step=2)(outer)
  wait_store(L - 2, 0)
  wait_store(L - 1, 1)


@jax.jit
def _embedding_sc(batch, weight):
  idxl = pl.kernel(
      _stage_idx_body,
      out_type=jax.ShapeDtypeStruct((B * L,), jnp.int32),
      mesh=_MESH,
      scratch_types=[
          pltpu.VMEM((8, BSLAB), jnp.int32),
          pltpu.VMEM((2, BSLAB), jnp.int32),
      ],
      compiler_params=pltpu.CompilerParams(use_tc_tiling_on_sc=True),
  )(batch.T)
  out = pl.kernel(
      _gather_body,
      out_type=jax.ShapeDtypeStruct((L, B, EMBED_DIM), jnp.float32),
      mesh=_MESH,
      scratch_types=[
          pltpu.VMEM((2, BSLAB), jnp.int32),
          pltpu.VMEM((2, BSLAB, EMBED_DIM), jnp.float32),
          pltpu.SemaphoreType.DMA,
          pltpu.SemaphoreType.DMA,
          pltpu.SemaphoreType.DMA,
          pltpu.SemaphoreType.DMA,
      ],
      compiler_params=pltpu.CompilerParams(use_tc_tiling_on_sc=False),
  )(idxl, weight)
  return out.transpose(1, 0, 2)


def kernel(batch, weight):
  return _embedding_sc(batch, weight)
